# Initial kernel scaffold; baseline (speedup 1.0000x reference)
import jax
import jax.numpy as jnp
from jax.experimental import pallas as pl


def kernel(input_seq, word_embedding, pe):
    raise NotImplementedError("write your pallas kernel here")


# speedup vs baseline: 4.2804x; 4.2804x over previous
import functools

import jax
import jax.numpy as jnp
from jax import lax
from jax.experimental import pallas as pl
from jax.experimental.pallas import tpu as pltpu
from jax.experimental.pallas import tpu_sc as plsc

VOCAB = 100000
EMBED = 128
MAXLEN = 200
NSEQ = 4096
PAD = 0

NC = 2
NS = 16
NW = NC * NS
SEQ_PER_W = NSEQ // NW
HALF = MAXLEN // 2
LANES = 16


def _sc_embed(seq3, table, pe2):
    mesh = plsc.VectorSubcoreMesh(
        core_axis_name="c", subcore_axis_name="s", num_cores=NC, num_subcores=NS
    )

    @functools.partial(
        pl.kernel,
        out_type=jax.ShapeDtypeStruct((NSEQ, MAXLEN, EMBED), jnp.float32),
        mesh=mesh,
        scratch_types=[
            pltpu.VMEM((2, HALF), jnp.int32),
            pltpu.VMEM((MAXLEN, EMBED), jnp.float32),
            pltpu.VMEM((MAXLEN, EMBED), jnp.float32),
            pltpu.SemaphoreType.DMA,
        ],
    )
    def body(seq_hbm, table_hbm, pe_hbm, out_hbm, idx_v, rows_v, pe_v, sem):
        wid = lax.axis_index("s") * NC + lax.axis_index("c")
        base = wid * SEQ_PER_W
        pltpu.sync_copy(pe_hbm, pe_v)

        def per_seq(i, _):
            s = base + i
            pltpu.sync_copy(seq_hbm.at[s], idx_v)
            g0 = pltpu.async_copy(
                table_hbm.at[idx_v.at[0]], rows_v.at[pl.ds(0, HALF)], sem
            )
            g1 = pltpu.async_copy(
                table_hbm.at[idx_v.at[1]], rows_v.at[pl.ds(HALF, HALF)], sem
            )
            g0.wait()
            g1.wait()

            def add_row(r, _):
                for cc in range(EMBED // LANES):
                    sl = pl.ds(cc * LANES, LANES)
                    plsc.addupdate(rows_v.at[r, sl], pe_v[r, sl])
                return 0

            lax.fori_loop(0, MAXLEN, add_row, 0)
            pltpu.sync_copy(rows_v, out_hbm.at[s])
            return 0

        lax.fori_loop(0, SEQ_PER_W, per_seq, 0)

    return body(seq3, table, pe2)


def _mask_body(x_ref, o_ref):
    o_ref[...] = x_ref[...] == PAD


_mask_call = pl.pallas_call(
    _mask_body,
    out_shape=jax.ShapeDtypeStruct((NSEQ, MAXLEN), jnp.bool_),
    grid=(16,),
    in_specs=[pl.BlockSpec((NSEQ // 16, MAXLEN), lambda i: (i, 0))],
    out_specs=pl.BlockSpec((NSEQ // 16, MAXLEN), lambda i: (i, 0)),
)


@jax.jit
def kernel(input_seq, word_embedding, pe):
    seq = input_seq.astype(jnp.int32)
    seq3 = seq.reshape(NSEQ, 2, HALF)
    pe2 = pe.reshape(MAXLEN, EMBED)
    in_embed = _sc_embed(seq3, word_embedding, pe2)
    mask = _mask_call(seq)
    return in_embed, mask


# speedup vs baseline: 7.5452x; 1.7627x over previous
import functools

import jax
import jax.numpy as jnp
from jax import lax
from jax.experimental import pallas as pl
from jax.experimental.pallas import tpu as pltpu
from jax.experimental.pallas import tpu_sc as plsc

VOCAB = 100000
EMBED = 128
MAXLEN = 200
NSEQ = 4096
PAD = 0

NC = 2
NS = 16
NW = NC * NS
SEQ_PER_W = NSEQ // NW
NPAIR = SEQ_PER_W // 2
HALF = MAXLEN // 2
LANES = 16


def _sc_embed(seq3, table, pe2):
    mesh = plsc.VectorSubcoreMesh(
        core_axis_name="c", subcore_axis_name="s", num_cores=NC, num_subcores=NS
    )

    @functools.partial(
        pl.kernel,
        out_type=jax.ShapeDtypeStruct((NSEQ, MAXLEN, EMBED), jnp.float32),
        mesh=mesh,
        scratch_types=[
            pltpu.VMEM((SEQ_PER_W, 2, HALF), jnp.int32),
            pltpu.VMEM((2, MAXLEN, EMBED), jnp.float32),
            pltpu.VMEM((MAXLEN, EMBED), jnp.float32),
            pltpu.SemaphoreType.DMA,
            pltpu.SemaphoreType.DMA,
            pltpu.SemaphoreType.DMA,
            pltpu.SemaphoreType.DMA,
        ],
    )
    def body(seq_hbm, table_hbm, pe_hbm, out_hbm, idx_v, rows_v, pe_v,
             gsem0, gsem1, osem0, osem1):
        wid = lax.axis_index("s") * NC + lax.axis_index("c")
        base = wid * SEQ_PER_W
        pltpu.sync_copy(pe_hbm, pe_v)
        pltpu.sync_copy(seq_hbm.at[pl.ds(base, SEQ_PER_W)], idx_v)

        def fire_gather(slot, i, gsem):
            pltpu.async_copy(
                table_hbm.at[idx_v.at[i, 0]], rows_v.at[slot, pl.ds(0, HALF)], gsem
            )
            pltpu.async_copy(
                table_hbm.at[idx_v.at[i, 1]], rows_v.at[slot, pl.ds(HALF, HALF)], gsem
            )

        def wait_gather(slot, i, gsem):
            pltpu.make_async_copy(
                table_hbm.at[idx_v.at[i, 0]], rows_v.at[slot, pl.ds(0, HALF)], gsem
            ).wait()
            pltpu.make_async_copy(
                table_hbm.at[idx_v.at[i, 1]], rows_v.at[slot, pl.ds(HALF, HALF)], gsem
            ).wait()

        def add_pe(slot):
            def add_row(r, _):
                for cc in range(EMBED // LANES):
                    sl = pl.ds(cc * LANES, LANES)
                    plsc.addupdate(rows_v.at[slot, r, sl], pe_v[r, sl])
                return 0

            lax.fori_loop(0, MAXLEN, add_row, 0)

        fire_gather(0, 0, gsem0)

        def per_pair(p, _):
            i0 = 2 * p
            i1 = i0 + 1
            s0 = base + i0
            s1 = s0 + 1

            @pl.when(p > 0)
            def _():
                pltpu.make_async_copy(rows_v.at[1], out_hbm.at[s1 - 2], osem1).wait()

            fire_gather(1, i1, gsem1)

            wait_gather(0, i0, gsem0)
            add_pe(0)
            pltpu.async_copy(rows_v.at[0], out_hbm.at[s0], osem0)

            @pl.when(p < NPAIR - 1)
            def _():
                pltpu.make_async_copy(rows_v.at[0], out_hbm.at[s0], osem0).wait()
                fire_gather(0, i0 + 2, gsem0)

            wait_gather(1, i1, gsem1)
            add_pe(1)
            pltpu.async_copy(rows_v.at[1], out_hbm.at[s1], osem1)
            return 0

        lax.fori_loop(0, NPAIR, per_pair, 0)

        pltpu.make_async_copy(
            rows_v.at[0], out_hbm.at[base + SEQ_PER_W - 2], osem0
        ).wait()
        pltpu.make_async_copy(
            rows_v.at[1], out_hbm.at[base + SEQ_PER_W - 1], osem1
        ).wait()

    return body(seq3, table, pe2)


def _mask_body(x_ref, o_ref):
    o_ref[...] = x_ref[...] == PAD


_mask_call = pl.pallas_call(
    _mask_body,
    out_shape=jax.ShapeDtypeStruct((NSEQ, MAXLEN), jnp.bool_),
    grid=(16,),
    in_specs=[pl.BlockSpec((NSEQ // 16, MAXLEN), lambda i: (i, 0))],
    out_specs=pl.BlockSpec((NSEQ // 16, MAXLEN), lambda i: (i, 0)),
)


@jax.jit
def kernel(input_seq, word_embedding, pe):
    seq = input_seq.astype(jnp.int32)
    seq3 = seq.reshape(NSEQ, 2, HALF)
    pe2 = pe.reshape(MAXLEN, EMBED)
    in_embed = _sc_embed(seq3, word_embedding, pe2)
    mask = _mask_call(seq)
    return in_embed, mask


# speedup vs baseline: 9.0168x; 1.1950x over previous
import functools

import jax
import jax.numpy as jnp
from jax import lax
from jax.experimental import pallas as pl
from jax.experimental.pallas import tpu as pltpu
from jax.experimental.pallas import tpu_sc as plsc

VOCAB = 100000
EMBED = 128
MAXLEN = 200
NSEQ = 4096
PAD = 0

NC = 2
NS = 16
NW = NC * NS
SEQ_PER_W = NSEQ // NW
HALF = MAXLEN // 2
LANES = 16


def _sc_embed(seq3, table, pe2):
    mesh = plsc.VectorSubcoreMesh(
        core_axis_name="c", subcore_axis_name="s", num_cores=NC, num_subcores=NS
    )

    @functools.partial(
        pl.kernel,
        out_type=jax.ShapeDtypeStruct((NSEQ, MAXLEN, EMBED), jnp.float32),
        mesh=mesh,
        scratch_types=[
            pltpu.VMEM((3, 2, HALF), jnp.int32),
            pltpu.VMEM((3, MAXLEN, EMBED), jnp.float32),
            pltpu.VMEM((MAXLEN, EMBED), jnp.float32),
            pltpu.SemaphoreType.DMA,
            pltpu.SemaphoreType.DMA,
            pltpu.SemaphoreType.DMA,
            pltpu.SemaphoreType.DMA,
            pltpu.SemaphoreType.DMA,
            pltpu.SemaphoreType.DMA,
            pltpu.SemaphoreType.DMA,
            pltpu.SemaphoreType.DMA,
            pltpu.SemaphoreType.DMA,
        ],
    )
    def body(seq_hbm, table_hbm, pe_hbm, out_hbm, idx_v, rows_v, pe_v,
             gsem0, gsem1, gsem2, osem0, osem1, osem2, isem0, isem1, isem2):
        gsems = (gsem0, gsem1, gsem2)
        osems = (osem0, osem1, osem2)
        isems = (isem0, isem1, isem2)
        wid = lax.axis_index("s") * NC + lax.axis_index("c")
        base = wid * SEQ_PER_W
        pltpu.sync_copy(pe_hbm, pe_v)

        def fire_idx(slot, i):
            pltpu.async_copy(seq_hbm.at[base + i], idx_v.at[slot], isems[slot])

        def wait_idx(slot, i):
            pltpu.make_async_copy(
                seq_hbm.at[base + i], idx_v.at[slot], isems[slot]
            ).wait()

        def fire_gather(slot, i):
            del i
            pltpu.async_copy(
                table_hbm.at[idx_v.at[slot, 0]], rows_v.at[slot, pl.ds(0, HALF)],
                gsems[slot],
            )
            pltpu.async_copy(
                table_hbm.at[idx_v.at[slot, 1]], rows_v.at[slot, pl.ds(HALF, HALF)],
                gsems[slot],
            )

        def wait_gather(slot, i):
            del i
            pltpu.make_async_copy(
                table_hbm.at[idx_v.at[slot, 0]], rows_v.at[slot, pl.ds(0, HALF)],
                gsems[slot],
            ).wait()
            pltpu.make_async_copy(
                table_hbm.at[idx_v.at[slot, 1]], rows_v.at[slot, pl.ds(HALF, HALF)],
                gsems[slot],
            ).wait()

        def add_pe(slot):
            @plsc.parallel_loop(0, MAXLEN, step=1, unroll=4)
            def _(r):
                for cc in range(EMBED // LANES):
                    sl = pl.ds(cc * LANES, LANES)
                    plsc.addupdate(rows_v.at[slot, r, sl], pe_v[r, sl])

        def wait_store(slot, s):
            pltpu.make_async_copy(rows_v.at[slot], out_hbm.at[s], osems[slot]).wait()

        pltpu.sync_copy(seq_hbm.at[base], idx_v.at[0])
        pltpu.sync_copy(seq_hbm.at[base + 1], idx_v.at[1])
        pltpu.sync_copy(seq_hbm.at[base + 2], idx_v.at[2])
        fire_gather(0, 0)
        fire_gather(1, 1)

        @pl.loop(0, SEQ_PER_W - 2, step=3)
        def _(g):
            for b in range(3):
                i = g + b
                s = base + i
                b2 = (b + 2) % 3
                wait_gather(b, i)
                @pl.when(i + 3 < SEQ_PER_W)
                def _pf():
                    fire_idx(b, i + 3)
                add_pe(b)
                if b == 0:
                    @pl.when(g > 0)
                    def _w():
                        wait_store(b2, s - 1)
                        wait_idx(b2, i + 2)
                else:
                    wait_idx(b2, i + 2)
                    wait_store(b2, s - 1)
                fire_gather(b2, i + 2)
                pltpu.async_copy(rows_v.at[b], out_hbm.at[s], osems[b])

        i = SEQ_PER_W - 2
        wait_gather(0, i)
        add_pe(0)
        wait_store(2, base + i - 1)
        pltpu.async_copy(rows_v.at[0], out_hbm.at[base + i], osems[0])
        wait_gather(1, i + 1)
        add_pe(1)
        pltpu.async_copy(rows_v.at[1], out_hbm.at[base + i + 1], osems[1])
        wait_store(0, base + i)
        wait_store(1, base + i + 1)

    return body(seq3, table, pe2)


def _mask_body(x_ref, o_ref):
    o_ref[...] = x_ref[...] == PAD


_mask_call = pl.pallas_call(
    _mask_body,
    out_shape=jax.ShapeDtypeStruct((NSEQ, MAXLEN), jnp.bool_),
    grid=(16,),
    in_specs=[pl.BlockSpec((NSEQ // 16, MAXLEN), lambda i: (i, 0))],
    out_specs=pl.BlockSpec((NSEQ // 16, MAXLEN), lambda i: (i, 0)),
)


@jax.jit
def kernel(input_seq, word_embedding, pe):
    seq = input_seq.astype(jnp.int32)
    seq3 = seq.reshape(NSEQ, 2, HALF)
    pe2 = pe.reshape(MAXLEN, EMBED)
    in_embed = _sc_embed(seq3, word_embedding, pe2)
    mask = _mask_call(seq)
    return in_embed, mask
